# baseline (speedup 1.0000x reference)
import functools

import jax
import jax.numpy as jnp
from jax import lax
from jax.experimental import pallas as pl
from jax.experimental.pallas import tpu as pltpu
from jax.experimental.pallas import tpu_sc as plsc

_N = 20000
_NS = 16
_CH = 1280
_NP = _NS * _CH
_SL = _CH // 16
_K = 100
_NEG = float("-inf")
_BIG = 3.0e7


def _shuf(v, perm):
    return lax.gather(
        v,
        perm[:, None],
        lax.GatherDimensionNumbers(
            offset_dims=(), collapsed_slice_dims=(0,), start_index_map=(0,)
        ),
        slice_sizes=(1,),
        mode=lax.GatherScatterMode.PROMISE_IN_BOUNDS,
    )


def _tree(op, v, li):
    for sh in (8, 4, 2, 1):
        v = op(v, _shuf(v, jnp.bitwise_xor(li, sh)))
    return v


def _nms_body(inp_hbm, out_hbm,
              px1, py1, px2, py2, d0, d1, d2, d3, sc,
              bx1, by1, bx2, by2, ar, sw,
              all16, recv, outb, shared):
    s = lax.axis_index("s")
    c = lax.axis_index("c")
    base = s * _CH
    basef = base.astype(jnp.float32)
    li = lax.iota(jnp.int32, 16)
    lif = li.astype(jnp.float32)
    idx00 = basef + lif

    for f, dst in enumerate((px1, py1, px2, py2, d0, d1, d2, d3, sc)):
        pltpu.sync_copy(inp_hbm.at[pl.ds(f * _NP + base, _CH)], dst)

    def dec(i, _):
        o = pl.ds(i * 16, 16)
        x1 = px1[o]
        y1 = py1[o]
        x2 = px2[o]
        y2 = py2[o]
        w = x2 - x1 + 1.0
        h = y2 - y1 + 1.0
        cx = x1 + 0.5 * w
        cy = y1 + 0.5 * h
        dx = d0[o] / 10.0
        dy = d1[o] / 10.0
        dw = jnp.minimum(d2[o] / 5.0, 4.0)
        dh = jnp.minimum(d3[o] / 5.0, 4.0)
        pcx = dx * w + cx
        pcy = dy * h + cy
        pw = jnp.exp(dw) * w
        ph = jnp.exp(dh) * h
        nx1 = jnp.clip(pcx - 0.5 * pw, 0.0, 1023.0)
        ny1 = jnp.clip(pcy - 0.5 * ph, 0.0, 1023.0)
        nx2 = jnp.clip(pcx + 0.5 * pw - 1.0, 0.0, 1023.0)
        ny2 = jnp.clip(pcy + 0.5 * ph - 1.0, 0.0, 1023.0)
        bx1[o] = nx1
        by1[o] = ny1
        bx2[o] = nx2
        by2[o] = ny2
        ar[o] = (nx2 - nx1 + 1.0) * (ny2 - ny1 + 1.0)
        sv = sc[o]
        sw[o] = jnp.where(sv > 0.05, sv, _NEG)
        return 0

    lax.fori_loop(0, _SL, dec, 0)

    _UNROLL = 8

    def pass_fn(wx1, wy1, wx2, wy2, war):
        def step(i, carry):
            m, vx = carry
            for u in range(_UNROLL):
                o = pl.ds(i * (16 * _UNROLL) + u * 16, 16)
                swv = sw[o]
                ix1 = jnp.maximum(wx1, bx1[o])
                iy1 = jnp.maximum(wy1, by1[o])
                ix2 = jnp.minimum(wx2, bx2[o])
                iy2 = jnp.minimum(wy2, by2[o])
                iw = jnp.maximum(ix2 - ix1 + 1.0, 0.0)
                ih = jnp.maximum(iy2 - iy1 + 1.0, 0.0)
                inter = iw * ih
                iou = inter / (war + ar[o] - inter)
                swv = jnp.where(iou > 0.5, _NEG, swv)
                sw[o] = swv
                idxg = idx00 + (i * (16 * _UNROLL) + u * 16).astype(jnp.float32)
                upd = swv > m
                m = jnp.where(upd, swv, m)
                vx = jnp.where(upd, idxg, vx)
            return (m, vx)

        m0 = jnp.full((16,), _NEG, jnp.float32)
        return lax.fori_loop(0, _SL // _UNROLL, step, (m0, idx00))

    fake = jnp.full((16,), _BIG, jnp.float32)
    one = jnp.full((16,), 1.0, jnp.float32)
    m, vx = pass_fn(fake, fake, -fake, -fake, one)

    def outer(k, carry):
        m, vx = carry
        lm = _tree(jnp.maximum, m, li)
        lidx = _tree(jnp.minimum, jnp.where(m == lm, vx, _BIG), li)
        offb = (lidx - basef).astype(jnp.int32)
        gx1 = plsc.load_gather(bx1, [offb])
        gy1 = plsc.load_gather(by1, [offb])
        gx2 = plsc.load_gather(bx2, [offb])
        gy2 = plsc.load_gather(by2, [offb])
        gsc = plsc.load_gather(sc, [offb])
        gar = plsc.load_gather(ar, [offb])
        rec = jnp.where(li == 0, lm,
              jnp.where(li == 1, lidx,
              jnp.where(li == 2, gx1,
              jnp.where(li == 3, gy1,
              jnp.where(li == 4, gx2,
              jnp.where(li == 5, gy2,
              jnp.where(li == 6, gsc, gar)))))))
        recv[...] = rec
        buf = (k % 2) * 256
        pltpu.sync_copy(recv, shared.at[pl.ds(buf + s * 16, 16)])
        plsc.subcore_barrier()
        pltpu.sync_copy(shared.at[pl.ds(buf, 256)], all16)
        maxv = plsc.load_gather(all16, [li * 16])
        idxv = plsc.load_gather(all16, [li * 16 + 1])
        mg = _tree(jnp.maximum, maxv, li)
        cand = jnp.where(maxv == mg, idxv, _BIG)
        bidx = _tree(jnp.minimum, cand, li)
        tstar = _tree(
            jnp.minimum, jnp.where(cand == bidx, lif, _BIG), li
        ).astype(jnp.int32)
        gb = tstar * 16
        wx1 = plsc.load_gather(all16, [gb + 2])
        wy1 = plsc.load_gather(all16, [gb + 3])
        wx2 = plsc.load_gather(all16, [gb + 4])
        wy2 = plsc.load_gather(all16, [gb + 5])
        wsc = plsc.load_gather(all16, [gb + 6])
        war = plsc.load_gather(all16, [gb + 7])
        orec = jnp.where(li == 0, wx1,
               jnp.where(li == 1, wy1,
               jnp.where(li == 2, wx2,
               jnp.where(li == 3, wy2,
               jnp.where(li == 4, wsc, 0.0)))))
        plsc.store_scatter(outb, [k * 16 + li], orec)
        return pass_fn(wx1, wy1, wx2, wy2, war)

    lax.fori_loop(0, _K, outer, (m, vx))

    @pl.when(jnp.logical_and(s == 0, c == 0))
    def _():
        pltpu.sync_copy(outb, out_hbm)


@functools.cache
def _get_nms():
    return pl.kernel(
        _nms_body,
        out_type=jax.ShapeDtypeStruct((_K * 16,), jnp.float32),
        mesh=plsc.VectorSubcoreMesh(
            core_axis_name="c", subcore_axis_name="s", num_cores=2, num_subcores=16
        ),
        scratch_types=(
            [pltpu.VMEM((_CH,), jnp.float32)] * 15
            + [
                pltpu.VMEM((256,), jnp.float32),
                pltpu.VMEM((16,), jnp.float32),
                pltpu.VMEM((_K * 16,), jnp.float32),
                pltpu.VMEM_SHARED((512,), jnp.float32),
            ]
        ),
        compiler_params=pltpu.CompilerParams(needs_layout_passes=False),
    )


@jax.jit
def kernel(proposals, box_regression, scores):
    pt = jnp.pad(proposals, ((0, _NP - _N), (0, 0))).T
    bt = jnp.pad(box_regression, ((0, _NP - _N), (0, 0))).T
    st = jnp.pad(scores, (0, _NP - _N))[None]
    inp = jnp.concatenate([pt, bt, st], axis=0).reshape(-1)
    out = _get_nms()(inp)
    return out.reshape(_K, 16)[:, :5]


# speedup vs baseline: 2.3126x; 2.3126x over previous
import functools

import jax
import jax.numpy as jnp
from jax import lax
from jax.experimental import pallas as pl
from jax.experimental.pallas import tpu as pltpu
from jax.experimental.pallas import tpu_sc as plsc

_N = 20000
_NS = 16
_CH = 1280
_NP = _NS * _CH
_SL = _CH // 16
_K = 100
_NEG = float("-inf")
_BIG = 3.0e7


def _shuf(v, perm):
    return lax.gather(
        v,
        perm[:, None],
        lax.GatherDimensionNumbers(
            offset_dims=(), collapsed_slice_dims=(0,), start_index_map=(0,)
        ),
        slice_sizes=(1,),
        mode=lax.GatherScatterMode.PROMISE_IN_BOUNDS,
    )


def _tree(op, v, li):
    for sh in (8, 4, 2, 1):
        v = op(v, _shuf(v, jnp.bitwise_xor(li, sh)))
    return v


def _nms_body(inp_hbm, out_hbm,
              px1, py1, px2, py2, d0, d1, d2, d3, sc,
              bx1, by1, bx2, by2, ar, sw,
              all16, recv, outb, shared):
    s = lax.axis_index("s")
    c = lax.axis_index("c")
    base = s * _CH
    basef = base.astype(jnp.float32)
    li = lax.iota(jnp.int32, 16)
    lif = li.astype(jnp.float32)
    idx00 = basef + lif

    for f, dst in enumerate((px1, py1, px2, py2, d0, d1, d2, d3, sc)):
        pltpu.sync_copy(inp_hbm.at[pl.ds(f * _NP + base, _CH)], dst)

    def dec(i, _):
        o = pl.ds(i * 16, 16)
        x1 = px1[o]
        y1 = py1[o]
        x2 = px2[o]
        y2 = py2[o]
        w = x2 - x1 + 1.0
        h = y2 - y1 + 1.0
        cx = x1 + 0.5 * w
        cy = y1 + 0.5 * h
        dx = d0[o] / 10.0
        dy = d1[o] / 10.0
        dw = jnp.minimum(d2[o] / 5.0, 4.0)
        dh = jnp.minimum(d3[o] / 5.0, 4.0)
        pcx = dx * w + cx
        pcy = dy * h + cy
        pw = jnp.exp(dw) * w
        ph = jnp.exp(dh) * h
        nx1 = jnp.clip(pcx - 0.5 * pw, 0.0, 1023.0)
        ny1 = jnp.clip(pcy - 0.5 * ph, 0.0, 1023.0)
        nx2 = jnp.clip(pcx + 0.5 * pw - 1.0, 0.0, 1023.0)
        ny2 = jnp.clip(pcy + 0.5 * ph - 1.0, 0.0, 1023.0)
        bx1[o] = nx1
        by1[o] = ny1
        bx2[o] = nx2
        by2[o] = ny2
        ar[o] = (nx2 - nx1 + 1.0) * (ny2 - ny1 + 1.0)
        sv = sc[o]
        sw[o] = jnp.where(sv > 0.05, sv, _NEG)
        return 0

    lax.fori_loop(0, _SL, dec, 0)

    _UNROLL = 5

    def pass_fn(wx1, wy1, wx2, wy2, war):
        def step(i, carry):
            m, vx = carry
            for u in range(_UNROLL):
                o = pl.ds(i * (16 * _UNROLL) + u * 16, 16)
                swv = sw[o]
                ix1 = jnp.maximum(wx1, bx1[o])
                iy1 = jnp.maximum(wy1, by1[o])
                ix2 = jnp.minimum(wx2, bx2[o])
                iy2 = jnp.minimum(wy2, by2[o])
                iw = jnp.maximum(ix2 - ix1 + 1.0, 0.0)
                ih = jnp.maximum(iy2 - iy1 + 1.0, 0.0)
                inter = iw * ih
                iou = inter / (war + ar[o] - inter)
                swv = jnp.where(iou > 0.5, _NEG, swv)
                sw[o] = swv
                idxg = idx00 + (i * (16 * _UNROLL) + u * 16).astype(jnp.float32)
                upd = swv > m
                m = jnp.where(upd, swv, m)
                vx = jnp.where(upd, idxg, vx)
            return (m, vx)

        m0 = jnp.full((16,), _NEG, jnp.float32)
        return lax.fori_loop(0, _SL // _UNROLL, step, (m0, idx00))

    fake = jnp.full((16,), _BIG, jnp.float32)
    one = jnp.full((16,), 1.0, jnp.float32)
    m, vx = pass_fn(fake, fake, -fake, -fake, one)

    def outer(k, carry):
        m, vx = carry
        lm = _tree(jnp.maximum, m, li)
        lidx = _tree(jnp.minimum, jnp.where(m == lm, vx, _BIG), li)
        offb = (lidx - basef).astype(jnp.int32)
        gx1 = plsc.load_gather(bx1, [offb])
        gy1 = plsc.load_gather(by1, [offb])
        gx2 = plsc.load_gather(bx2, [offb])
        gy2 = plsc.load_gather(by2, [offb])
        gsc = plsc.load_gather(sc, [offb])
        gar = plsc.load_gather(ar, [offb])
        rec = jnp.where(li == 0, lm,
              jnp.where(li == 1, lidx,
              jnp.where(li == 2, gx1,
              jnp.where(li == 3, gy1,
              jnp.where(li == 4, gx2,
              jnp.where(li == 5, gy2,
              jnp.where(li == 6, gsc, gar)))))))
        recv[...] = rec
        buf = (k % 2) * 256
        pltpu.sync_copy(recv, shared.at[pl.ds(buf + s * 16, 16)])
        plsc.subcore_barrier()
        pltpu.sync_copy(shared.at[pl.ds(buf, 256)], all16)
        maxv = plsc.load_gather(all16, [li * 16])
        idxv = plsc.load_gather(all16, [li * 16 + 1])
        mg = _tree(jnp.maximum, maxv, li)
        cand = jnp.where(maxv == mg, idxv, _BIG)
        bidx = _tree(jnp.minimum, cand, li)
        tstar = _tree(
            jnp.minimum, jnp.where(cand == bidx, lif, _BIG), li
        ).astype(jnp.int32)
        gb = tstar * 16
        wx1 = plsc.load_gather(all16, [gb + 2])
        wy1 = plsc.load_gather(all16, [gb + 3])
        wx2 = plsc.load_gather(all16, [gb + 4])
        wy2 = plsc.load_gather(all16, [gb + 5])
        wsc = plsc.load_gather(all16, [gb + 6])
        war = plsc.load_gather(all16, [gb + 7])
        orec = jnp.where(li == 0, wx1,
               jnp.where(li == 1, wy1,
               jnp.where(li == 2, wx2,
               jnp.where(li == 3, wy2,
               jnp.where(li == 4, wsc, 0.0)))))
        plsc.store_scatter(outb, [k * 16 + li], orec)
        return pass_fn(wx1, wy1, wx2, wy2, war)

    lax.fori_loop(0, _K, outer, (m, vx))

    @pl.when(jnp.logical_and(s == 0, c == 0))
    def _():
        pltpu.sync_copy(outb, out_hbm)


@functools.cache
def _get_nms():
    return pl.kernel(
        _nms_body,
        out_type=jax.ShapeDtypeStruct((_K * 16,), jnp.float32),
        mesh=plsc.VectorSubcoreMesh(
            core_axis_name="c", subcore_axis_name="s", num_cores=2, num_subcores=16
        ),
        scratch_types=(
            [pltpu.VMEM((_CH,), jnp.float32)] * 15
            + [
                pltpu.VMEM((256,), jnp.float32),
                pltpu.VMEM((16,), jnp.float32),
                pltpu.VMEM((_K * 16,), jnp.float32),
                pltpu.VMEM_SHARED((512,), jnp.float32),
            ]
        ),
        compiler_params=pltpu.CompilerParams(needs_layout_passes=False),
    )


@jax.jit
def kernel(proposals, box_regression, scores):
    pt = jnp.pad(proposals, ((0, _NP - _N), (0, 0))).T
    bt = jnp.pad(box_regression, ((0, _NP - _N), (0, 0))).T
    st = jnp.pad(scores, (0, _NP - _N))[None]
    inp = jnp.concatenate([pt, bt, st], axis=0).reshape(-1)
    out = _get_nms()(inp)
    return out.reshape(_K, 16)[:, :5]


# speedup vs baseline: 2.6405x; 1.1418x over previous
import functools

import jax
import jax.numpy as jnp
from jax import lax
from jax.experimental import pallas as pl
from jax.experimental.pallas import tpu as pltpu
from jax.experimental.pallas import tpu_sc as plsc

_N = 20000
_NS = 16
_CH = 1280
_NP = _NS * _CH
_SL = _CH // 16
_K = 100
_NEG = float("-inf")
_BIG = 3.0e7
_UNROLL = 4


def _shuf(v, perm):
    return lax.gather(
        v,
        perm[:, None],
        lax.GatherDimensionNumbers(
            offset_dims=(), collapsed_slice_dims=(0,), start_index_map=(0,)
        ),
        slice_sizes=(1,),
        mode=lax.GatherScatterMode.PROMISE_IN_BOUNDS,
    )


def _better(av, ai, bv, bi):
    return (av > bv) | ((av == bv) & (ai < bi))


def _nms_body(inp_hbm, out_hbm,
              px1, py1, px2, py2, d0, d1, d2, d3, sc,
              bx1, by1, bx2, by2, ar, sw,
              all16, recv, cnt, outb, shared):
    s = lax.axis_index("s")
    c = lax.axis_index("c")
    base = s * _CH
    basef = base.astype(jnp.float32)
    li = lax.iota(jnp.int32, 16)
    idx00 = basef + li.astype(jnp.float32)

    for f, dst in enumerate((px1, py1, px2, py2, d0, d1, d2, d3, sc)):
        pltpu.sync_copy(inp_hbm.at[pl.ds(f * _NP + base, _CH)], dst)

    def dec(i, _):
        o = pl.ds(i * 16, 16)
        x1 = px1[o]
        y1 = py1[o]
        x2 = px2[o]
        y2 = py2[o]
        w = x2 - x1 + 1.0
        h = y2 - y1 + 1.0
        cx = x1 + 0.5 * w
        cy = y1 + 0.5 * h
        dx = d0[o] / 10.0
        dy = d1[o] / 10.0
        dw = jnp.minimum(d2[o] / 5.0, 4.0)
        dh = jnp.minimum(d3[o] / 5.0, 4.0)
        pcx = dx * w + cx
        pcy = dy * h + cy
        pw = jnp.exp(dw) * w
        ph = jnp.exp(dh) * h
        nx1 = jnp.clip(pcx - 0.5 * pw, 0.0, 1023.0)
        ny1 = jnp.clip(pcy - 0.5 * ph, 0.0, 1023.0)
        nx2 = jnp.clip(pcx + 0.5 * pw - 1.0, 0.0, 1023.0)
        ny2 = jnp.clip(pcy + 0.5 * ph - 1.0, 0.0, 1023.0)
        bx1[o] = nx1
        by1[o] = ny1
        bx2[o] = nx2
        by2[o] = ny2
        ar[o] = (nx2 - nx1 + 1.0) * (ny2 - ny1 + 1.0)
        sv = sc[o]
        sw[o] = jnp.where(sv > 0.05, sv, _NEG)
        return 0

    lax.fori_loop(0, _SL, dec, 0)

    def pass_fn(w1x1, w1y1, w1x2, w1y2, w1ar,
                w2x1, w2y1, w2x2, w2y2, w2ar):
        def step(i, carry):
            m1, v1, m2, v2 = carry
            for u in range(_UNROLL):
                off = i * (16 * _UNROLL) + u * 16
                o = pl.ds(off, 16)
                swv = sw[o]
                ex1 = bx1[o]
                ey1 = by1[o]
                ex2 = bx2[o]
                ey2 = by2[o]
                ear = ar[o]
                ia1 = jnp.maximum(w1x1, ex1)
                ib1 = jnp.maximum(w1y1, ey1)
                ic1 = jnp.minimum(w1x2, ex2)
                id1 = jnp.minimum(w1y2, ey2)
                iw1 = jnp.maximum(ic1 - ia1 + 1.0, 0.0)
                ih1 = jnp.maximum(id1 - ib1 + 1.0, 0.0)
                in1 = iw1 * ih1
                iou1 = in1 / (w1ar + ear - in1)
                ia2 = jnp.maximum(w2x1, ex1)
                ib2 = jnp.maximum(w2y1, ey1)
                ic2 = jnp.minimum(w2x2, ex2)
                id2 = jnp.minimum(w2y2, ey2)
                iw2 = jnp.maximum(ic2 - ia2 + 1.0, 0.0)
                ih2 = jnp.maximum(id2 - ib2 + 1.0, 0.0)
                in2 = iw2 * ih2
                iou2 = in2 / (w2ar + ear - in2)
                swv = jnp.where((iou1 > 0.5) | (iou2 > 0.5), _NEG, swv)
                sw[o] = swv
                idxg = idx00 + off.astype(jnp.float32)
                c1 = swv > m1
                c2 = swv > m2
                m2 = jnp.where(c1, m1, jnp.where(c2, swv, m2))
                v2 = jnp.where(c1, v1, jnp.where(c2, idxg, v2))
                m1 = jnp.where(c1, swv, m1)
                v1 = jnp.where(c1, idxg, v1)
            return (m1, v1, m2, v2)

        m0 = jnp.full((16,), _NEG, jnp.float32)
        return lax.fori_loop(
            0, _SL // _UNROLL, step, (m0, idx00, m0, idx00)
        )

    fake = jnp.full((16,), _BIG, jnp.float32)
    one = jnp.full((16,), 1.0, jnp.float32)
    quads = pass_fn(fake, fake, -fake, -fake, one,
                    fake, fake, -fake, -fake, one)

    def round_body(carry):
        count, r, m1, v1, m2, v2 = carry
        for sh in (8, 4, 2, 1):
            perm = jnp.bitwise_xor(li, sh)
            b1v = _shuf(m1, perm)
            b1i = _shuf(v1, perm)
            b2v = _shuf(m2, perm)
            b2i = _shuf(v2, perm)
            bw = _better(m1, v1, b1v, b1i)
            t1v = jnp.where(bw, m1, b1v)
            t1i = jnp.where(bw, v1, b1i)
            lv = jnp.where(bw, b1v, m1)
            lvi = jnp.where(bw, b1i, v1)
            wv = jnp.where(bw, m2, b2v)
            wvi = jnp.where(bw, v2, b2i)
            b2w = _better(lv, lvi, wv, wvi)
            m1, v1 = t1v, t1i
            m2 = jnp.where(b2w, lv, wv)
            v2 = jnp.where(b2w, lvi, wvi)
        off1 = (v1 - basef).astype(jnp.int32)
        off2 = (v2 - basef).astype(jnp.int32)
        g1 = [plsc.load_gather(ref, [off1]) for ref in (bx1, by1, bx2, by2, sc, ar)]
        g2 = [plsc.load_gather(ref, [off2]) for ref in (bx1, by1, bx2, by2, sc, ar)]
        vals = [m1, v1, m2, v2] + g1 + g2
        rec = vals[15]
        for ci in range(14, -1, -1):
            rec = jnp.where(li == ci, vals[ci], rec)
        recv[...] = rec
        buf = (r % 2) * 256
        pltpu.sync_copy(recv, shared.at[pl.ds(buf + s * 16, 16)])
        plsc.subcore_barrier()
        pltpu.sync_copy(shared.at[pl.ds(buf, 256)], all16)
        a1v = plsc.load_gather(all16, [li * 16])
        a1i = plsc.load_gather(all16, [li * 16 + 1])
        a2v = plsc.load_gather(all16, [li * 16 + 2])
        a2i = plsc.load_gather(all16, [li * 16 + 3])
        a1p = li * 16 + 4
        a2p = li * 16 + 10
        for sh in (8, 4, 2, 1):
            perm = jnp.bitwise_xor(li, sh)
            b1v = _shuf(a1v, perm)
            b1i = _shuf(a1i, perm)
            b1p = _shuf(a1p, perm)
            b2v = _shuf(a2v, perm)
            b2i = _shuf(a2i, perm)
            b2p = _shuf(a2p, perm)
            bw = _better(a1v, a1i, b1v, b1i)
            t1v = jnp.where(bw, a1v, b1v)
            t1i = jnp.where(bw, a1i, b1i)
            t1p = jnp.where(bw, a1p, b1p)
            lv = jnp.where(bw, b1v, a1v)
            lvi = jnp.where(bw, b1i, a1i)
            lp = jnp.where(bw, b1p, a1p)
            wv = jnp.where(bw, a2v, b2v)
            wvi = jnp.where(bw, a2i, b2i)
            wp = jnp.where(bw, a2p, b2p)
            b2w = _better(lv, lvi, wv, wvi)
            a1v, a1i, a1p = t1v, t1i, t1p
            a2v = jnp.where(b2w, lv, wv)
            a2i = jnp.where(b2w, lvi, wvi)
            a2p = jnp.where(b2w, lp, wp)
        w1x1 = plsc.load_gather(all16, [a1p])
        w1y1 = plsc.load_gather(all16, [a1p + 1])
        w1x2 = plsc.load_gather(all16, [a1p + 2])
        w1y2 = plsc.load_gather(all16, [a1p + 3])
        w1sc = plsc.load_gather(all16, [a1p + 4])
        w1ar = plsc.load_gather(all16, [a1p + 5])
        w2x1 = plsc.load_gather(all16, [a2p])
        w2y1 = plsc.load_gather(all16, [a2p + 1])
        w2x2 = plsc.load_gather(all16, [a2p + 2])
        w2y2 = plsc.load_gather(all16, [a2p + 3])
        w2sc = plsc.load_gather(all16, [a2p + 4])
        w2ar = plsc.load_gather(all16, [a2p + 5])
        jx1 = jnp.maximum(w1x1, w2x1)
        jy1 = jnp.maximum(w1y1, w2y1)
        jx2 = jnp.minimum(w1x2, w2x2)
        jy2 = jnp.minimum(w1y2, w2y2)
        jw = jnp.maximum(jx2 - jx1 + 1.0, 0.0)
        jh = jnp.maximum(jy2 - jy1 + 1.0, 0.0)
        jin = jw * jh
        jiou = jin / (w1ar + w2ar - jin)
        acceptv = (
            (jiou <= 0.5)
            & (a2v > _NEG)
            & jnp.broadcast_to(count <= _K - 2, (16,))
        )
        orec1 = jnp.where(li == 0, w1x1,
                jnp.where(li == 1, w1y1,
                jnp.where(li == 2, w1x2,
                jnp.where(li == 3, w1y2,
                jnp.where(li == 4, w1sc, 0.0)))))
        plsc.store_scatter(outb, [count * 16 + li], orec1)
        orec2 = jnp.where(li == 0, w2x1,
                jnp.where(li == 1, w2y1,
                jnp.where(li == 2, w2x2,
                jnp.where(li == 3, w2y2,
                jnp.where(li == 4, w2sc, 0.0)))))
        plsc.store_scatter(outb, [(count + 1) * 16 + li], orec2, mask=acceptv)
        count = count + 1 + acceptv.astype(jnp.int32)[0]
        sx1 = jnp.where(acceptv, w2x1, fake)
        sy1 = jnp.where(acceptv, w2y1, fake)
        sx2 = jnp.where(acceptv, w2x2, -fake)
        sy2 = jnp.where(acceptv, w2y2, -fake)
        sar = jnp.where(acceptv, w2ar, one)
        m1, v1, m2, v2 = pass_fn(w1x1, w1y1, w1x2, w1y2, w1ar,
                                 sx1, sy1, sx2, sy2, sar)
        return (count, r + 1, m1, v1, m2, v2)

    init = (jnp.int32(0), jnp.int32(0)) + quads
    lax.while_loop(lambda cr: cr[0] < _K, round_body, init)

    @pl.when(jnp.logical_and(s == 0, c == 0))
    def _():
        pltpu.sync_copy(outb, out_hbm)


@functools.cache
def _get_nms():
    return pl.kernel(
        _nms_body,
        out_type=jax.ShapeDtypeStruct((_K * 16,), jnp.float32),
        mesh=plsc.VectorSubcoreMesh(
            core_axis_name="c", subcore_axis_name="s", num_cores=2, num_subcores=16
        ),
        scratch_types=(
            [pltpu.VMEM((_CH,), jnp.float32)] * 15
            + [
                pltpu.VMEM((256,), jnp.float32),
                pltpu.VMEM((16,), jnp.float32),
                pltpu.VMEM((16,), jnp.int32),
                pltpu.VMEM((_K * 16,), jnp.float32),
                pltpu.VMEM_SHARED((512,), jnp.float32),
            ]
        ),
        compiler_params=pltpu.CompilerParams(needs_layout_passes=False),
    )


@jax.jit
def kernel(proposals, box_regression, scores):
    pt = jnp.pad(proposals, ((0, _NP - _N), (0, 0))).T
    bt = jnp.pad(box_regression, ((0, _NP - _N), (0, 0))).T
    st = jnp.pad(scores, (0, _NP - _N))[None]
    inp = jnp.concatenate([pt, bt, st], axis=0).reshape(-1)
    out = _get_nms()(inp)
    return out.reshape(_K, 16)[:, :5]
